# baseline (speedup 1.0000x reference)
import functools

import jax
import jax.numpy as jnp
from jax import lax
from jax.experimental import pallas as pl
from jax.experimental.pallas import tpu as pltpu
from jax.experimental.pallas import tpu_sc as plsc

NW = 32
CH = 128
GRP = 8
ZR = 64
CPW0 = 160
CPW1 = 0


def _sc_mesh():
    return plsc.VectorSubcoreMesh(core_axis_name="c", subcore_axis_name="s")


def _sc_degree(dst2, npad, cpw):
    rpt = npad // 16

    @functools.partial(
        pl.kernel,
        out_type=jax.ShapeDtypeStruct((2, npad), jnp.float32),
        mesh=_sc_mesh(),
        scratch_types=[
            pltpu.VMEM((cpw, CH), jnp.int32),
            pltpu.VMEM((CH,), jnp.float32),
            pltpu.VMEM((rpt,), jnp.float32),
            pltpu.VMEM_SHARED((npad,), jnp.float32),
        ],
    )
    def k(dst_h, out_h, didx, ones, stage, hist):
        c = lax.axis_index("c")
        s = lax.axis_index("s")
        wid = s * 2 + c
        sl = pl.ds(s * rpt, rpt)

        def fill(i, _):
            stage[pl.ds(i * 16, 16)] = jnp.zeros((16,), jnp.float32)
            return _

        lax.fori_loop(0, rpt // 16, fill, None)

        def fill1(i, _):
            ones[pl.ds(i * 16, 16)] = jnp.ones((16,), jnp.float32)
            return _

        lax.fori_loop(0, CH // 16, fill1, None)
        pltpu.sync_copy(stage, hist.at[sl])
        pltpu.sync_copy(dst_h.at[pl.ds(wid * cpw, cpw)], didx)
        plsc.subcore_barrier()

        def step(j, _):
            pltpu.sync_copy(ones, hist.at[didx.at[j]], add=True)
            return _

        lax.fori_loop(0, cpw, step, None)
        plsc.subcore_barrier()
        pltpu.sync_copy(hist.at[sl], stage)
        pltpu.sync_copy(stage, out_h.at[c, sl])

    return k(dst2)


def _sc_scatter(src2, dst2, y, npad):
    rpt = npad // 16

    @functools.partial(
        pl.kernel,
        out_type=jax.ShapeDtypeStruct((2, npad, 128), jnp.float32),
        mesh=_sc_mesh(),
        scratch_types=[
            pltpu.VMEM((GRP, CH), jnp.int32),
            pltpu.VMEM((GRP, CH), jnp.int32),
            pltpu.VMEM((GRP, CH), jnp.int32),
            pltpu.VMEM((GRP, CH), jnp.int32),
            pltpu.VMEM((CH, 128), jnp.float32),
            pltpu.VMEM((CH, 128), jnp.float32),
            pltpu.VMEM((ZR, 128), jnp.float32),
            pltpu.VMEM_SHARED((npad, 128), jnp.float32),
            pltpu.SemaphoreType.DMA,
            pltpu.SemaphoreType.DMA,
        ],
    )
    def k(src_h, dst_h, y_h, out_h, siA, siB, diA, diB, rows0, rows1, stage,
          acc, sem0, sem1):
        c = lax.axis_index("c")
        s = lax.axis_index("s")
        base = s * rpt
        si = [siA, siB]
        di = [diA, diB]
        rows = [rows0, rows1]
        sems = [sem0, sem1]

        def fill(i, _):
            def fcol(j, __):
                stage[i, pl.ds(j * 16, 16)] = jnp.zeros((16,), jnp.float32)
                return __

            return lax.fori_loop(0, 128 // 16, fcol, _)

        lax.fori_loop(0, ZR, fill, None)

        def zcp(t, _):
            pltpu.sync_copy(stage, acc.at[pl.ds(base + t * ZR, ZR)])
            return _

        lax.fori_loop(0, rpt // ZR, zcp, None)
        plsc.subcore_barrier()

        def edge_loop(row0, cpw):
            assert cpw % (2 * GRP) == 0
            nsteps = cpw // (2 * GRP)
            pltpu.sync_copy(src_h.at[pl.ds(row0, GRP)], siA)
            pltpu.sync_copy(dst_h.at[pl.ds(row0, GRP)], diA)
            pltpu.async_copy(y_h.at[siA.at[0]], rows0, sem0)

            def outer(u, _):
                rbase = row0 + u * 2 * GRP
                pltpu.sync_copy(src_h.at[pl.ds(rbase + GRP, GRP)], siB)
                pltpu.sync_copy(dst_h.at[pl.ds(rbase + GRP, GRP)], diB)
                for k_ in range(2 * GRP):
                    cur = k_ % 2
                    nxt = 1 - cur
                    g_cur, r_cur = divmod(k_, GRP)
                    if k_ + 1 < 2 * GRP:
                        g_n, r_n = divmod(k_ + 1, GRP)
                        pltpu.async_copy(
                            y_h.at[si[g_n].at[r_n]], rows[nxt], sems[nxt])
                    else:
                        @pl.when(u + 1 < nsteps)
                        def _pre():
                            pltpu.sync_copy(
                                src_h.at[pl.ds(rbase + 2 * GRP, GRP)], siA)
                            pltpu.sync_copy(
                                dst_h.at[pl.ds(rbase + 2 * GRP, GRP)], diA)
                            pltpu.async_copy(
                                y_h.at[siA.at[0]], rows[nxt], sems[nxt])

                    pltpu.make_async_copy(
                        y_h.at[si[g_cur].at[r_cur]], rows[cur], sems[cur]).wait()
                    pltpu.sync_copy(
                        rows[cur], acc.at[di[g_cur].at[r_cur]], add=True)
                return _

            lax.fori_loop(0, nsteps, outer, None)

        @pl.when(c == 0)
        def _core0():
            edge_loop(s * CPW0, CPW0)

        if CPW1:
            @pl.when(c == 1)
            def _core1():
                edge_loop(16 * CPW0 + s * CPW1, CPW1)

        plsc.subcore_barrier()

        def wout(t, _):
            sl = pl.ds(base + t * ZR, ZR)
            pltpu.sync_copy(acc.at[sl], stage)
            pltpu.sync_copy(stage, out_h.at[c, sl])
            return _

        lax.fori_loop(0, rpt // ZR, wout, None)

    return k(src2, dst2, y)


def _row_block_specs(npad, r):
    degp_spec = pl.BlockSpec((2, r), lambda i: (0, i))
    mat_spec = pl.BlockSpec((r, 128), lambda i: (i, 0))
    acc_spec = pl.BlockSpec((2, r, 128), lambda i: (0, i, 0))
    w_spec = pl.BlockSpec((128, 128), lambda i: (0, 0))
    b_spec = pl.BlockSpec((1, 128), lambda i: (0, 0))
    return degp_spec, mat_spec, acc_spec, w_spec, b_spec


def _tc_lin(degp, xp, w):
    npad = xp.shape[0]
    r = 1024
    degp_spec, mat_spec, acc_spec, w_spec, b_spec = _row_block_specs(npad, r)

    def body(degp_ref, x_ref, w_ref, o_ref):
        dinv = lax.rsqrt(degp_ref[0] + degp_ref[1] + 1.0)
        y = jnp.dot(x_ref[...], w_ref[...], preferred_element_type=jnp.float32)
        o_ref[...] = y * dinv[:, None]

    return pl.pallas_call(
        body,
        grid=(npad // r,),
        in_specs=[degp_spec, mat_spec, w_spec],
        out_specs=mat_spec,
        out_shape=jax.ShapeDtypeStruct((npad, 128), jnp.float32),
    )(degp, xp, w)


def _tc_mid(degp, acc, y1, b1, w2):
    npad = y1.shape[0]
    r = 1024
    degp_spec, mat_spec, acc_spec, w_spec, b_spec = _row_block_specs(npad, r)

    def body(degp_ref, acc_ref, y1_ref, b1_ref, w2_ref, o_ref):
        dinv = lax.rsqrt(degp_ref[0] + degp_ref[1] + 1.0)[:, None]
        a = acc_ref[0] + acc_ref[1] + y1_ref[...]
        h = jnp.maximum(dinv * a + b1_ref[...], 0.0)
        y = jnp.dot(h, w2_ref[...], preferred_element_type=jnp.float32)
        o_ref[...] = y * dinv

    return pl.pallas_call(
        body,
        grid=(npad // r,),
        in_specs=[degp_spec, acc_spec, mat_spec, b_spec, w_spec],
        out_specs=mat_spec,
        out_shape=jax.ShapeDtypeStruct((npad, 128), jnp.float32),
    )(degp, acc, y1, b1, w2)


def _tc_out(degp, acc, y2, b2):
    npad = y2.shape[0]
    r = 1024
    degp_spec, mat_spec, acc_spec, w_spec, b_spec = _row_block_specs(npad, r)

    def body(degp_ref, acc_ref, y2_ref, b2_ref, o_ref):
        dinv = lax.rsqrt(degp_ref[0] + degp_ref[1] + 1.0)[:, None]
        a = acc_ref[0] + acc_ref[1] + y2_ref[...]
        o_ref[...] = dinv * a + b2_ref[...]

    return pl.pallas_call(
        body,
        grid=(npad // r,),
        in_specs=[degp_spec, acc_spec, mat_spec, b_spec],
        out_specs=mat_spec,
        out_shape=jax.ShapeDtypeStruct((npad, 128), jnp.float32),
    )(degp, acc, y2, b2)


def kernel(x, edge_index, W1, b1, W2, b2):
    n, d = x.shape
    e = edge_index.shape[1]
    rpt = -(-n // (16 * ZR)) * ZR
    npad = rpt * 16
    tot = 16 * (CPW0 + CPW1)
    ep = tot * CH
    assert ep >= e and tot % NW == 0
    cpw_u = tot // NW
    assert cpw_u % GRP == 0

    src = edge_index[0].astype(jnp.int32)
    dst = edge_index[1].astype(jnp.int32)
    pad = ep - e
    src2 = jnp.concatenate([src, jnp.zeros((pad,), jnp.int32)]).reshape(tot, CH)
    dst2 = jnp.concatenate([dst, jnp.full((pad,), npad - 8, jnp.int32)]).reshape(
        tot, CH)
    xp = jnp.pad(x, ((0, npad - n), (0, 0)))
    b1r = b1.reshape(1, d)
    b2r = b2.reshape(1, d)

    degp = _sc_degree(dst2, npad, cpw_u)
    y1 = _tc_lin(degp, xp, W1)
    acc1 = _sc_scatter(src2, dst2, y1, npad)
    y2 = _tc_mid(degp, acc1, y1, b1r, W2)
    acc2 = _sc_scatter(src2, dst2, y2, npad)
    out = _tc_out(degp, acc2, y2, b2r)
    return out[:n]


# speedup vs baseline: 2.5644x; 2.5644x over previous
import functools

import jax
import jax.numpy as jnp
from jax import lax
from jax.experimental import pallas as pl
from jax.experimental.pallas import tpu as pltpu
from jax.experimental.pallas import tpu_sc as plsc

NW = 32
CH = 128
GRP = 8
ZR = 64
CPW0 = 128
CPW1 = 32


def _sc_mesh():
    return plsc.VectorSubcoreMesh(core_axis_name="c", subcore_axis_name="s")


def _sc_degree(dst2, npad, cpw):
    rpt = npad // 16

    @functools.partial(
        pl.kernel,
        out_type=jax.ShapeDtypeStruct((2, npad), jnp.float32),
        mesh=_sc_mesh(),
        scratch_types=[
            pltpu.VMEM((cpw, CH), jnp.int32),
            pltpu.VMEM((CH,), jnp.float32),
            pltpu.VMEM((rpt,), jnp.float32),
            pltpu.VMEM_SHARED((npad,), jnp.float32),
        ],
    )
    def k(dst_h, out_h, didx, ones, stage, hist):
        c = lax.axis_index("c")
        s = lax.axis_index("s")
        wid = s * 2 + c
        sl = pl.ds(s * rpt, rpt)

        def fill(i, _):
            stage[pl.ds(i * 16, 16)] = jnp.zeros((16,), jnp.float32)
            return _

        lax.fori_loop(0, rpt // 16, fill, None)

        def fill1(i, _):
            ones[pl.ds(i * 16, 16)] = jnp.ones((16,), jnp.float32)
            return _

        lax.fori_loop(0, CH // 16, fill1, None)
        pltpu.sync_copy(stage, hist.at[sl])
        pltpu.sync_copy(dst_h.at[pl.ds(wid * cpw, cpw)], didx)
        plsc.subcore_barrier()

        def step(j, _):
            pltpu.sync_copy(ones, hist.at[didx.at[j]], add=True)
            return _

        lax.fori_loop(0, cpw, step, None)
        plsc.subcore_barrier()
        pltpu.sync_copy(hist.at[sl], stage)
        pltpu.sync_copy(stage, out_h.at[c, sl])

    return k(dst2)


def _sc_scatter(src2, dst2, y, npad):
    rpt = npad // 16

    @functools.partial(
        pl.kernel,
        out_type=jax.ShapeDtypeStruct((2, npad, 128), jnp.float32),
        mesh=_sc_mesh(),
        scratch_types=[
            pltpu.VMEM((GRP, CH), jnp.int32),
            pltpu.VMEM((GRP, CH), jnp.int32),
            pltpu.VMEM((GRP, CH), jnp.int32),
            pltpu.VMEM((GRP, CH), jnp.int32),
            pltpu.VMEM((CH, 128), jnp.float32),
            pltpu.VMEM((CH, 128), jnp.float32),
            pltpu.VMEM((ZR, 128), jnp.float32),
            pltpu.VMEM_SHARED((npad, 128), jnp.float32),
            pltpu.SemaphoreType.DMA,
            pltpu.SemaphoreType.DMA,
        ],
    )
    def k(src_h, dst_h, y_h, out_h, siA, siB, diA, diB, rows0, rows1, stage,
          acc, sem0, sem1):
        c = lax.axis_index("c")
        s = lax.axis_index("s")
        base = s * rpt
        si = [siA, siB]
        di = [diA, diB]
        rows = [rows0, rows1]
        sems = [sem0, sem1]

        def fill(i, _):
            def fcol(j, __):
                stage[i, pl.ds(j * 16, 16)] = jnp.zeros((16,), jnp.float32)
                return __

            return lax.fori_loop(0, 128 // 16, fcol, _)

        lax.fori_loop(0, ZR, fill, None)

        def zcp(t, _):
            pltpu.sync_copy(stage, acc.at[pl.ds(base + t * ZR, ZR)])
            return _

        lax.fori_loop(0, rpt // ZR, zcp, None)
        plsc.subcore_barrier()

        def edge_loop(row0, cpw):
            assert cpw % (2 * GRP) == 0
            nsteps = cpw // (2 * GRP)
            pltpu.sync_copy(src_h.at[pl.ds(row0, GRP)], siA)
            pltpu.sync_copy(dst_h.at[pl.ds(row0, GRP)], diA)
            pltpu.async_copy(y_h.at[siA.at[0]], rows0, sem0)

            def outer(u, _):
                rbase = row0 + u * 2 * GRP
                pltpu.sync_copy(src_h.at[pl.ds(rbase + GRP, GRP)], siB)
                pltpu.sync_copy(dst_h.at[pl.ds(rbase + GRP, GRP)], diB)
                for k_ in range(2 * GRP):
                    cur = k_ % 2
                    nxt = 1 - cur
                    g_cur, r_cur = divmod(k_, GRP)
                    if k_ + 1 < 2 * GRP:
                        g_n, r_n = divmod(k_ + 1, GRP)
                        pltpu.async_copy(
                            y_h.at[si[g_n].at[r_n]], rows[nxt], sems[nxt])
                    else:
                        @pl.when(u + 1 < nsteps)
                        def _pre():
                            pltpu.sync_copy(
                                src_h.at[pl.ds(rbase + 2 * GRP, GRP)], siA)
                            pltpu.sync_copy(
                                dst_h.at[pl.ds(rbase + 2 * GRP, GRP)], diA)
                            pltpu.async_copy(
                                y_h.at[siA.at[0]], rows[nxt], sems[nxt])

                    pltpu.make_async_copy(
                        y_h.at[si[g_cur].at[r_cur]], rows[cur], sems[cur]).wait()
                    pltpu.sync_copy(
                        rows[cur], acc.at[di[g_cur].at[r_cur]], add=True)
                return _

            lax.fori_loop(0, nsteps, outer, None)

        @pl.when(c == 0)
        def _core0():
            edge_loop(s * CPW0, CPW0)

        if CPW1:
            @pl.when(c == 1)
            def _core1():
                edge_loop(16 * CPW0 + s * CPW1, CPW1)

        plsc.subcore_barrier()

        def wout(t, _):
            sl = pl.ds(base + t * ZR, ZR)
            pltpu.sync_copy(acc.at[sl], stage)
            pltpu.sync_copy(stage, out_h.at[c, sl])
            return _

        lax.fori_loop(0, rpt // ZR, wout, None)

    return k(src2, dst2, y)


def _row_block_specs(npad, r):
    degp_spec = pl.BlockSpec((2, r), lambda i: (0, i))
    mat_spec = pl.BlockSpec((r, 128), lambda i: (i, 0))
    acc_spec = pl.BlockSpec((2, r, 128), lambda i: (0, i, 0))
    w_spec = pl.BlockSpec((128, 128), lambda i: (0, 0))
    b_spec = pl.BlockSpec((1, 128), lambda i: (0, 0))
    return degp_spec, mat_spec, acc_spec, w_spec, b_spec


def _tc_lin(degp, xp, w):
    npad = xp.shape[0]
    r = 1024
    degp_spec, mat_spec, acc_spec, w_spec, b_spec = _row_block_specs(npad, r)

    def body(degp_ref, x_ref, w_ref, o_ref):
        dinv = lax.rsqrt(degp_ref[0] + degp_ref[1] + 1.0)
        y = jnp.dot(x_ref[...], w_ref[...], preferred_element_type=jnp.float32)
        o_ref[...] = y * dinv[:, None]

    return pl.pallas_call(
        body,
        grid=(npad // r,),
        in_specs=[degp_spec, mat_spec, w_spec],
        out_specs=mat_spec,
        out_shape=jax.ShapeDtypeStruct((npad, 128), jnp.float32),
    )(degp, xp, w)


def _tc_mid(degp, acc, y1, b1, w2):
    npad = y1.shape[0]
    r = 1024
    degp_spec, mat_spec, acc_spec, w_spec, b_spec = _row_block_specs(npad, r)

    def body(degp_ref, acc_ref, y1_ref, b1_ref, w2_ref, o_ref):
        dinv = lax.rsqrt(degp_ref[0] + degp_ref[1] + 1.0)[:, None]
        a = acc_ref[0] + acc_ref[1] + y1_ref[...]
        h = jnp.maximum(dinv * a + b1_ref[...], 0.0)
        y = jnp.dot(h, w2_ref[...], preferred_element_type=jnp.float32)
        o_ref[...] = y * dinv

    return pl.pallas_call(
        body,
        grid=(npad // r,),
        in_specs=[degp_spec, acc_spec, mat_spec, b_spec, w_spec],
        out_specs=mat_spec,
        out_shape=jax.ShapeDtypeStruct((npad, 128), jnp.float32),
    )(degp, acc, y1, b1, w2)


def _tc_out(degp, acc, y2, b2):
    npad = y2.shape[0]
    r = 1024
    degp_spec, mat_spec, acc_spec, w_spec, b_spec = _row_block_specs(npad, r)

    def body(degp_ref, acc_ref, y2_ref, b2_ref, o_ref):
        dinv = lax.rsqrt(degp_ref[0] + degp_ref[1] + 1.0)[:, None]
        a = acc_ref[0] + acc_ref[1] + y2_ref[...]
        o_ref[...] = dinv * a + b2_ref[...]

    return pl.pallas_call(
        body,
        grid=(npad // r,),
        in_specs=[degp_spec, acc_spec, mat_spec, b_spec],
        out_specs=mat_spec,
        out_shape=jax.ShapeDtypeStruct((npad, 128), jnp.float32),
    )(degp, acc, y2, b2)


def kernel(x, edge_index, W1, b1, W2, b2):
    n, d = x.shape
    e = edge_index.shape[1]
    rpt = -(-n // (16 * ZR)) * ZR
    npad = rpt * 16
    tot = 16 * (CPW0 + CPW1)
    ep = tot * CH
    assert ep >= e and tot % NW == 0
    cpw_u = tot // NW
    assert cpw_u % GRP == 0

    src = edge_index[0].astype(jnp.int32)
    dst = edge_index[1].astype(jnp.int32)
    pad = ep - e
    spare = npad - n
    pidx = jnp.arange(pad, dtype=jnp.int32)
    psrc = pidx % jnp.int32(n)
    pdst = jnp.int32(n) + pidx % jnp.int32(spare)
    src2 = jnp.concatenate([src, psrc]).reshape(tot, CH)
    dst2 = jnp.concatenate([dst, pdst]).reshape(tot, CH)
    xp = jnp.pad(x, ((0, npad - n), (0, 0)))
    b1r = b1.reshape(1, d)
    b2r = b2.reshape(1, d)

    degp = _sc_degree(dst2, npad, cpw_u)
    y1 = _tc_lin(degp, xp, W1)
    acc1 = _sc_scatter(src2, dst2, y1, npad)
    y2 = _tc_mid(degp, acc1, y1, b1r, W2)
    acc2 = _sc_scatter(src2, dst2, y2, npad)
    out = _tc_out(degp, acc2, y2, b2r)
    return out[:n]


# speedup vs baseline: 3.5026x; 1.3659x over previous
import functools

import jax
import jax.numpy as jnp
from jax import lax
from jax.experimental import pallas as pl
from jax.experimental.pallas import tpu as pltpu
from jax.experimental.pallas import tpu_sc as plsc

NW = 32
CH = 128
GRP = 8
ZR = 64
CPW0 = 80
CPW1 = 80


def _sc_mesh():
    return plsc.VectorSubcoreMesh(core_axis_name="c", subcore_axis_name="s")


def _sc_degree(dst2, npad, cpw):
    rpt = npad // 16

    @functools.partial(
        pl.kernel,
        out_type=jax.ShapeDtypeStruct((2, npad), jnp.float32),
        mesh=_sc_mesh(),
        scratch_types=[
            pltpu.VMEM((cpw, CH), jnp.int32),
            pltpu.VMEM((CH,), jnp.float32),
            pltpu.VMEM((rpt,), jnp.float32),
            pltpu.VMEM_SHARED((npad,), jnp.float32),
        ],
    )
    def k(dst_h, out_h, didx, ones, stage, hist):
        c = lax.axis_index("c")
        s = lax.axis_index("s")
        wid = s * 2 + c
        sl = pl.ds(s * rpt, rpt)

        def fill(i, _):
            stage[pl.ds(i * 16, 16)] = jnp.zeros((16,), jnp.float32)
            return _

        lax.fori_loop(0, rpt // 16, fill, None)

        def fill1(i, _):
            ones[pl.ds(i * 16, 16)] = jnp.ones((16,), jnp.float32)
            return _

        lax.fori_loop(0, CH // 16, fill1, None)
        pltpu.sync_copy(stage, hist.at[sl])
        pltpu.sync_copy(dst_h.at[pl.ds(wid * cpw, cpw)], didx)
        plsc.subcore_barrier()

        def step(j, _):
            pltpu.sync_copy(ones, hist.at[didx.at[j]], add=True)
            return _

        lax.fori_loop(0, cpw, step, None)
        plsc.subcore_barrier()
        pltpu.sync_copy(hist.at[sl], stage)
        pltpu.sync_copy(stage, out_h.at[c, sl])

    return k(dst2)


def _sc_scatter(src2, dst2, y, npad):
    rpt = npad // 16

    @functools.partial(
        pl.kernel,
        out_type=jax.ShapeDtypeStruct((2, npad, 128), jnp.float32),
        mesh=_sc_mesh(),
        scratch_types=[
            pltpu.VMEM((GRP, CH), jnp.int32),
            pltpu.VMEM((GRP, CH), jnp.int32),
            pltpu.VMEM((GRP, CH), jnp.int32),
            pltpu.VMEM((GRP, CH), jnp.int32),
            pltpu.VMEM((CH, 128), jnp.float32),
            pltpu.VMEM((CH, 128), jnp.float32),
            pltpu.VMEM((ZR, 128), jnp.float32),
            pltpu.VMEM_SHARED((npad, 128), jnp.float32),
            pltpu.SemaphoreType.DMA,
            pltpu.SemaphoreType.DMA,
        ],
    )
    def k(src_h, dst_h, y_h, out_h, siA, siB, diA, diB, rows0, rows1, stage,
          acc, sem0, sem1):
        c = lax.axis_index("c")
        s = lax.axis_index("s")
        base = s * rpt
        si = [siA, siB]
        di = [diA, diB]
        rows = [rows0, rows1]
        sems = [sem0, sem1]

        def fill(i, _):
            def fcol(j, __):
                stage[i, pl.ds(j * 16, 16)] = jnp.zeros((16,), jnp.float32)
                return __

            return lax.fori_loop(0, 128 // 16, fcol, _)

        lax.fori_loop(0, ZR, fill, None)

        def zcp(t, _):
            pltpu.sync_copy(stage, acc.at[pl.ds(base + t * ZR, ZR)])
            return _

        lax.fori_loop(0, rpt // ZR, zcp, None)
        plsc.subcore_barrier()

        def edge_loop(row0, cpw):
            assert cpw % (2 * GRP) == 0
            nsteps = cpw // (2 * GRP)
            pltpu.sync_copy(src_h.at[pl.ds(row0, GRP)], siA)
            pltpu.sync_copy(dst_h.at[pl.ds(row0, GRP)], diA)
            pltpu.async_copy(y_h.at[siA.at[0]], rows0, sem0)

            def outer(u, _):
                rbase = row0 + u * 2 * GRP
                pltpu.sync_copy(src_h.at[pl.ds(rbase + GRP, GRP)], siB)
                pltpu.sync_copy(dst_h.at[pl.ds(rbase + GRP, GRP)], diB)
                for k_ in range(2 * GRP):
                    cur = k_ % 2
                    nxt = 1 - cur
                    g_cur, r_cur = divmod(k_, GRP)
                    if k_ + 1 < 2 * GRP:
                        g_n, r_n = divmod(k_ + 1, GRP)
                        pltpu.async_copy(
                            y_h.at[si[g_n].at[r_n]], rows[nxt], sems[nxt])
                    else:
                        @pl.when(u + 1 < nsteps)
                        def _pre():
                            pltpu.sync_copy(
                                src_h.at[pl.ds(rbase + 2 * GRP, GRP)], siA)
                            pltpu.sync_copy(
                                dst_h.at[pl.ds(rbase + 2 * GRP, GRP)], diA)
                            pltpu.async_copy(
                                y_h.at[siA.at[0]], rows[nxt], sems[nxt])

                    pltpu.make_async_copy(
                        y_h.at[si[g_cur].at[r_cur]], rows[cur], sems[cur]).wait()
                    pltpu.sync_copy(
                        rows[cur], acc.at[di[g_cur].at[r_cur]], add=True)
                return _

            lax.fori_loop(0, nsteps, outer, None)

        @pl.when(c == 0)
        def _core0():
            edge_loop(s * CPW0, CPW0)

        if CPW1:
            @pl.when(c == 1)
            def _core1():
                edge_loop(16 * CPW0 + s * CPW1, CPW1)

        plsc.subcore_barrier()

        def wout(t, _):
            sl = pl.ds(base + t * ZR, ZR)
            pltpu.sync_copy(acc.at[sl], stage)
            pltpu.sync_copy(stage, out_h.at[c, sl])
            return _

        lax.fori_loop(0, rpt // ZR, wout, None)

    return k(src2, dst2, y)


def _row_block_specs(npad, r):
    degp_spec = pl.BlockSpec((2, r), lambda i: (0, i))
    mat_spec = pl.BlockSpec((r, 128), lambda i: (i, 0))
    acc_spec = pl.BlockSpec((2, r, 128), lambda i: (0, i, 0))
    w_spec = pl.BlockSpec((128, 128), lambda i: (0, 0))
    b_spec = pl.BlockSpec((1, 128), lambda i: (0, 0))
    return degp_spec, mat_spec, acc_spec, w_spec, b_spec


def _tc_lin(degp, xp, w):
    npad = xp.shape[0]
    r = 1024
    degp_spec, mat_spec, acc_spec, w_spec, b_spec = _row_block_specs(npad, r)

    def body(degp_ref, x_ref, w_ref, o_ref):
        dinv = lax.rsqrt(degp_ref[0] + degp_ref[1] + 1.0)
        y = jnp.dot(x_ref[...], w_ref[...], preferred_element_type=jnp.float32)
        o_ref[...] = y * dinv[:, None]

    return pl.pallas_call(
        body,
        grid=(npad // r,),
        in_specs=[degp_spec, mat_spec, w_spec],
        out_specs=mat_spec,
        out_shape=jax.ShapeDtypeStruct((npad, 128), jnp.float32),
    )(degp, xp, w)


def _tc_mid(degp, acc, y1, b1, w2):
    npad = y1.shape[0]
    r = 1024
    degp_spec, mat_spec, acc_spec, w_spec, b_spec = _row_block_specs(npad, r)

    def body(degp_ref, acc_ref, y1_ref, b1_ref, w2_ref, o_ref):
        dinv = lax.rsqrt(degp_ref[0] + degp_ref[1] + 1.0)[:, None]
        a = acc_ref[0] + acc_ref[1] + y1_ref[...]
        h = jnp.maximum(dinv * a + b1_ref[...], 0.0)
        y = jnp.dot(h, w2_ref[...], preferred_element_type=jnp.float32)
        o_ref[...] = y * dinv

    return pl.pallas_call(
        body,
        grid=(npad // r,),
        in_specs=[degp_spec, acc_spec, mat_spec, b_spec, w_spec],
        out_specs=mat_spec,
        out_shape=jax.ShapeDtypeStruct((npad, 128), jnp.float32),
    )(degp, acc, y1, b1, w2)


def _tc_out(degp, acc, y2, b2):
    npad = y2.shape[0]
    r = 1024
    degp_spec, mat_spec, acc_spec, w_spec, b_spec = _row_block_specs(npad, r)

    def body(degp_ref, acc_ref, y2_ref, b2_ref, o_ref):
        dinv = lax.rsqrt(degp_ref[0] + degp_ref[1] + 1.0)[:, None]
        a = acc_ref[0] + acc_ref[1] + y2_ref[...]
        o_ref[...] = dinv * a + b2_ref[...]

    return pl.pallas_call(
        body,
        grid=(npad // r,),
        in_specs=[degp_spec, acc_spec, mat_spec, b_spec],
        out_specs=mat_spec,
        out_shape=jax.ShapeDtypeStruct((npad, 128), jnp.float32),
    )(degp, acc, y2, b2)


def kernel(x, edge_index, W1, b1, W2, b2):
    n, d = x.shape
    e = edge_index.shape[1]
    rpt = -(-n // (16 * ZR)) * ZR
    npad = rpt * 16
    tot = 16 * (CPW0 + CPW1)
    ep = tot * CH
    assert ep >= e and tot % NW == 0
    cpw_u = tot // NW
    assert cpw_u % GRP == 0

    src = edge_index[0].astype(jnp.int32)
    dst = edge_index[1].astype(jnp.int32)
    pad = ep - e
    spare = npad - n
    pidx = jnp.arange(pad, dtype=jnp.int32)
    psrc = pidx % jnp.int32(n)
    pdst = jnp.int32(n) + pidx % jnp.int32(spare)
    src2 = jnp.concatenate([src, psrc]).reshape(tot, CH)
    dst2 = jnp.concatenate([dst, pdst]).reshape(tot, CH)
    xp = jnp.pad(x, ((0, npad - n), (0, 0)))
    b1r = b1.reshape(1, d)
    b2r = b2.reshape(1, d)

    degp = _sc_degree(dst2, npad, cpw_u)
    y1 = _tc_lin(degp, xp, W1)
    acc1 = _sc_scatter(src2, dst2, y1, npad)
    y2 = _tc_mid(degp, acc1, y1, b1r, W2)
    acc2 = _sc_scatter(src2, dst2, y2, npad)
    out = _tc_out(degp, acc2, y2, b2r)
    return out[:n]
